# baseline (speedup 1.0000x reference)
import functools

import jax
import jax.numpy as jnp
from jax import lax
from jax.experimental import pallas as pl
from jax.experimental.pallas import tpu as pltpu
from jax.experimental.pallas import tpu_sc as plsc

N = 10000
NP = 10112
DH = 128
E = 160000
TILES = 16
K = 96
SLOTS = 2
NCHUNK = 108
EPT = NCHUNK * K
ROWS_PT = NP // TILES

@functools.cache
def _make_sc_agg():
    mesh = plsc.VectorSubcoreMesh(core_axis_name="c", subcore_axis_name="s")
    return pl.kernel(
        _sc_agg_body,
        out_type=(
            jax.ShapeDtypeStruct((NP, DH), jnp.float32),
            jax.ShapeDtypeStruct((NP, DH), jnp.float32),
        ),
        mesh=mesh,
        scratch_types=[
            pltpu.VMEM((EPT,), jnp.int32),
            pltpu.VMEM((NCHUNK, K), jnp.int32),
            pltpu.VMEM((SLOTS * K, DH), jnp.float32),
            pltpu.VMEM_SHARED((NP, DH), jnp.float32),
            pltpu.SemaphoreType.DMA,
            pltpu.SemaphoreType.DMA,
            pltpu.SemaphoreType.DMA,
            pltpu.SemaphoreType.DMA,
        ],
    )


def _sc_agg_body(h_lo, h_hi, srcp, dstp, m_lo, m_hi,
                 src_v, dst_v, ring, acc, gs0, gs1, ss0, ss1):
    gsem = (gs0, gs1)
    ssem = (ss0, ss1)
    c = lax.axis_index("c")
    s = lax.axis_index("s")
    base = s * ROWS_PT

    @pl.when(c == 0)
    def _():
        pltpu.sync_copy(h_lo.at[pl.ds(base, ROWS_PT)], acc.at[pl.ds(base, ROWS_PT)])

    @pl.when(c == 1)
    def _():
        pltpu.sync_copy(h_hi.at[pl.ds(base, ROWS_PT)], acc.at[pl.ds(base, ROWS_PT)])

    plsc.subcore_barrier()

    def edges(h_ref):
        pltpu.sync_copy(srcp.at[s], src_v)
        pltpu.sync_copy(dstp.at[s], dst_v)

        def slot_buf(slot):
            return ring.at[pl.ds(slot * K, K)]

        def gather(chunk, slot):
            pltpu.async_copy(
                h_ref.at[src_v.at[pl.ds(chunk * K, K)]], slot_buf(slot), gsem[slot])

        def scatter(chunk, slot):
            pltpu.async_copy(
                slot_buf(slot), acc.at[dst_v.at[chunk]], ssem[slot], add=True)

        def wait_gather(chunk, slot):
            pltpu.make_async_copy(
                h_ref.at[src_v.at[pl.ds(chunk * K, K)]], slot_buf(slot),
                gsem[slot]).wait()

        def wait_scatter(chunk, slot):
            pltpu.make_async_copy(
                slot_buf(slot), acc.at[dst_v.at[chunk]], ssem[slot]).wait()

        for slot in range(SLOTS):
            gather(slot, slot)

        def body(j, carry):
            cbase = SLOTS * j
            for slot in range(SLOTS):
                wait_gather(cbase + slot, slot)
                scatter(cbase + slot, slot)
            for slot in range(SLOTS):
                nxt = cbase + slot + SLOTS

                @pl.when(nxt < NCHUNK)
                def _(slot=slot, nxt=nxt):
                    wait_scatter(nxt - SLOTS, slot)
                    gather(nxt, slot)

            return carry

        lax.fori_loop(0, NCHUNK // SLOTS, body, 0)
        for slot in range(SLOTS):
            wait_scatter(NCHUNK - SLOTS + slot, slot)

    @pl.when(c == 0)
    def _():
        edges(h_lo)

    @pl.when(c == 1)
    def _():
        edges(h_hi)

    plsc.subcore_barrier()

    @pl.when(c == 0)
    def _():
        pltpu.sync_copy(acc.at[pl.ds(base, ROWS_PT)], m_lo.at[pl.ds(base, ROWS_PT)])

    @pl.when(c == 1)
    def _():
        pltpu.sync_copy(acc.at[pl.ds(base, ROWS_PT)], m_hi.at[pl.ds(base, ROWS_PT)])


def _mlp_body_split(mlo, mhi, w1, b1, w2, b2, olo, ohi):
    m = jnp.concatenate([mlo[...], mhi[...]], axis=1)
    t = jnp.maximum(jnp.dot(m, w1[...], preferred_element_type=jnp.float32) + b1[...], 0.0)
    h = jnp.dot(t, w2[...], preferred_element_type=jnp.float32) + b2[...]
    olo[...] = h[:, :DH]
    ohi[...] = h[:, DH:]


def _mlp_body_final(mlo, mhi, w1, b1, w2, b2, out):
    m = jnp.concatenate([mlo[...], mhi[...]], axis=1)
    t = jnp.maximum(jnp.dot(m, w1[...], preferred_element_type=jnp.float32) + b1[...], 0.0)
    out[...] = jnp.dot(t, w2[...], preferred_element_type=jnp.float32) + b2[...]


_BM = 1264
_W_SPECS = [
    pl.BlockSpec((2 * DH, 2 * DH), lambda i: (0, 0)),
    pl.BlockSpec((1, 2 * DH), lambda i: (0, 0)),
    pl.BlockSpec((2 * DH, 2 * DH), lambda i: (0, 0)),
    pl.BlockSpec((1, 2 * DH), lambda i: (0, 0)),
]

_mlp_split = pl.pallas_call(
    _mlp_body_split,
    grid=(NP // _BM,),
    in_specs=[
        pl.BlockSpec((_BM, DH), lambda i: (i, 0)),
        pl.BlockSpec((_BM, DH), lambda i: (i, 0)),
        *_W_SPECS,
    ],
    out_specs=[
        pl.BlockSpec((_BM, DH), lambda i: (i, 0)),
        pl.BlockSpec((_BM, DH), lambda i: (i, 0)),
    ],
    out_shape=[
        jax.ShapeDtypeStruct((NP, DH), jnp.float32),
        jax.ShapeDtypeStruct((NP, DH), jnp.float32),
    ],
)

_BMF = 1000
_mlp_final = pl.pallas_call(
    _mlp_body_final,
    grid=(N // _BMF,),
    in_specs=[
        pl.BlockSpec((_BMF, DH), lambda i: (i, 0)),
        pl.BlockSpec((_BMF, DH), lambda i: (i, 0)),
        *_W_SPECS,
    ],
    out_specs=pl.BlockSpec((_BMF, 2 * DH), lambda i: (i, 0)),
    out_shape=jax.ShapeDtypeStruct((N, 2 * DH), jnp.float32),
)


def kernel(x, edge_index, W1_0, b1_0, W2_0, b2_0, W1_1, b1_1, W2_1, b2_1,
           W1_2, b1_2, W2_2, b2_2):
    pad = TILES * EPT - E
    srcp = jnp.concatenate(
        [edge_index[0], jnp.zeros((pad,), jnp.int32)]).reshape(TILES, EPT)
    dstp = jnp.concatenate(
        [edge_index[1], jnp.full((pad,), N, jnp.int32)]).reshape(TILES, NCHUNK, K)

    h_lo = jnp.zeros((NP, DH), jnp.float32).at[:N].set(x[:, :DH])
    h_hi = jnp.zeros((NP, DH), jnp.float32).at[:N].set(x[:, DH:])

    params = [
        (W1_0, b1_0.reshape(1, -1), W2_0, b2_0.reshape(1, -1)),
        (W1_1, b1_1.reshape(1, -1), W2_1, b2_1.reshape(1, -1)),
        (W1_2, b1_2.reshape(1, -1), W2_2, b2_2.reshape(1, -1)),
    ]
    for i, (W1, b1, W2, b2) in enumerate(params):
        m_lo, m_hi = _make_sc_agg()(h_lo, h_hi, srcp, dstp)
        if i < 2:
            h_lo, h_hi = _mlp_split(m_lo, m_hi, W1, b1, W2, b2)
        else:
            return _mlp_final(m_lo, m_hi, W1, b1, W2, b2)


# speedup vs baseline: 1.5818x; 1.5818x over previous
import functools

import jax
import jax.numpy as jnp
from jax import lax
from jax.experimental import pallas as pl
from jax.experimental.pallas import tpu as pltpu
from jax.experimental.pallas import tpu_sc as plsc

N = 10000
NP = 10112
DH = 128
E = 160000
TILES = 16
K = 128
SLOTS = 2
NCHUNK = 80
EPT = NCHUNK * K
GRPS = (48, 32)
GMAX = max(GRPS)
ROWS_PT = NP // TILES

@functools.cache
def _make_sc_agg():
    mesh = plsc.VectorSubcoreMesh(core_axis_name="c", subcore_axis_name="s")
    return pl.kernel(
        _sc_agg_body,
        out_type=(
            jax.ShapeDtypeStruct((NP, DH), jnp.float32),
            jax.ShapeDtypeStruct((NP, DH), jnp.float32),
        ),
        mesh=mesh,
        scratch_types=[
            pltpu.VMEM((EPT,), jnp.int32),
            pltpu.VMEM((GMAX, K), jnp.int32),
            pltpu.VMEM((SLOTS * K, DH), jnp.float32),
            pltpu.VMEM_SHARED((NP, DH), jnp.float32),
            pltpu.SemaphoreType.DMA,
            pltpu.SemaphoreType.DMA,
        ],
    )


def _sc_agg_body(h_lo, h_hi, srcp, dstp, m_lo, m_hi,
                 src_v, dst_v, ring, acc, gs0, gs1):
    gsem = (gs0, gs1)
    c = lax.axis_index("c")
    s = lax.axis_index("s")
    base = s * ROWS_PT

    @pl.when(c == 0)
    def _():
        pltpu.sync_copy(h_lo.at[pl.ds(base, ROWS_PT)], acc.at[pl.ds(base, ROWS_PT)])

    @pl.when(c == 1)
    def _():
        pltpu.sync_copy(h_hi.at[pl.ds(base, ROWS_PT)], acc.at[pl.ds(base, ROWS_PT)])

    plsc.subcore_barrier()

    def edges(h_ref):
        pltpu.sync_copy(srcp.at[s], src_v)

        def slot_buf(slot):
            return ring.at[pl.ds(slot * K, K)]

        def gather(chunk, slot):
            pltpu.async_copy(
                h_ref.at[src_v.at[pl.ds(chunk * K, K)]], slot_buf(slot), gsem[slot])

        def wait_gather(chunk, slot):
            pltpu.make_async_copy(
                h_ref.at[src_v.at[pl.ds(chunk * K, K)]], slot_buf(slot),
                gsem[slot]).wait()

        gather(0, 0)
        goff = 0
        for gsize in GRPS:
            pairs = gsize // 2
            pltpu.sync_copy(dstp.at[s, pl.ds(goff, gsize)], dst_v.at[pl.ds(0, gsize)])

            def body(j, carry, goff=goff, pairs=pairs):
                a = goff + 2 * j
                b = a + 1
                gather(b, 1)
                wait_gather(a, 0)
                pltpu.sync_copy(slot_buf(0), acc.at[dst_v.at[2 * j]], add=True)

                @pl.when(a + 2 < NCHUNK)
                def _():
                    gather(a + 2, 0)

                wait_gather(b, 1)
                pltpu.sync_copy(slot_buf(1), acc.at[dst_v.at[2 * j + 1]], add=True)
                return carry

            lax.fori_loop(0, pairs, body, 0)
            goff += gsize

    @pl.when(c == 0)
    def _():
        edges(h_lo)

    @pl.when(c == 1)
    def _():
        edges(h_hi)

    plsc.subcore_barrier()

    @pl.when(c == 0)
    def _():
        pltpu.sync_copy(acc.at[pl.ds(base, ROWS_PT)], m_lo.at[pl.ds(base, ROWS_PT)])

    @pl.when(c == 1)
    def _():
        pltpu.sync_copy(acc.at[pl.ds(base, ROWS_PT)], m_hi.at[pl.ds(base, ROWS_PT)])


def _mlp_body_split(mlo, mhi, w1, b1, w2, b2, olo, ohi):
    m = jnp.concatenate([mlo[...], mhi[...]], axis=1)
    t = jnp.maximum(jnp.dot(m, w1[...], preferred_element_type=jnp.float32) + b1[...], 0.0)
    h = jnp.dot(t, w2[...], preferred_element_type=jnp.float32) + b2[...]
    olo[...] = h[:, :DH]
    ohi[...] = h[:, DH:]


def _mlp_body_final(mlo, mhi, w1, b1, w2, b2, out):
    m = jnp.concatenate([mlo[...], mhi[...]], axis=1)
    t = jnp.maximum(jnp.dot(m, w1[...], preferred_element_type=jnp.float32) + b1[...], 0.0)
    out[...] = jnp.dot(t, w2[...], preferred_element_type=jnp.float32) + b2[...]


_BM = 1264
_W_SPECS = [
    pl.BlockSpec((2 * DH, 2 * DH), lambda i: (0, 0)),
    pl.BlockSpec((1, 2 * DH), lambda i: (0, 0)),
    pl.BlockSpec((2 * DH, 2 * DH), lambda i: (0, 0)),
    pl.BlockSpec((1, 2 * DH), lambda i: (0, 0)),
]

_mlp_split = pl.pallas_call(
    _mlp_body_split,
    grid=(NP // _BM,),
    in_specs=[
        pl.BlockSpec((_BM, DH), lambda i: (i, 0)),
        pl.BlockSpec((_BM, DH), lambda i: (i, 0)),
        *_W_SPECS,
    ],
    out_specs=[
        pl.BlockSpec((_BM, DH), lambda i: (i, 0)),
        pl.BlockSpec((_BM, DH), lambda i: (i, 0)),
    ],
    out_shape=[
        jax.ShapeDtypeStruct((NP, DH), jnp.float32),
        jax.ShapeDtypeStruct((NP, DH), jnp.float32),
    ],
)

_BMF = 1000
_mlp_final = pl.pallas_call(
    _mlp_body_final,
    grid=(N // _BMF,),
    in_specs=[
        pl.BlockSpec((_BMF, DH), lambda i: (i, 0)),
        pl.BlockSpec((_BMF, DH), lambda i: (i, 0)),
        *_W_SPECS,
    ],
    out_specs=pl.BlockSpec((_BMF, 2 * DH), lambda i: (i, 0)),
    out_shape=jax.ShapeDtypeStruct((N, 2 * DH), jnp.float32),
)


def kernel(x, edge_index, W1_0, b1_0, W2_0, b2_0, W1_1, b1_1, W2_1, b2_1,
           W1_2, b1_2, W2_2, b2_2):
    pad = TILES * EPT - E
    srcp = jnp.concatenate(
        [edge_index[0], jnp.zeros((pad,), jnp.int32)]).reshape(TILES, EPT)
    dstp = jnp.concatenate(
        [edge_index[1], jnp.full((pad,), N, jnp.int32)]).reshape(TILES, NCHUNK, K)

    h_lo = jnp.zeros((NP, DH), jnp.float32).at[:N].set(x[:, :DH])
    h_hi = jnp.zeros((NP, DH), jnp.float32).at[:N].set(x[:, DH:])

    params = [
        (W1_0, b1_0.reshape(1, -1), W2_0, b2_0.reshape(1, -1)),
        (W1_1, b1_1.reshape(1, -1), W2_1, b2_1.reshape(1, -1)),
        (W1_2, b1_2.reshape(1, -1), W2_2, b2_2.reshape(1, -1)),
    ]
    for i, (W1, b1, W2, b2) in enumerate(params):
        m_lo, m_hi = _make_sc_agg()(h_lo, h_hi, srcp, dstp)
        if i < 2:
            h_lo, h_hi = _mlp_split(m_lo, m_hi, W1, b1, W2, b2)
        else:
            return _mlp_final(m_lo, m_hi, W1, b1, W2, b2)


# speedup vs baseline: 1.6131x; 1.0198x over previous
import functools

import jax
import jax.numpy as jnp
from jax import lax
from jax.experimental import pallas as pl
from jax.experimental.pallas import tpu as pltpu
from jax.experimental.pallas import tpu_sc as plsc

N = 10000
NP = 10112
DH = 128
E = 160000
TILES = 16
K = 128
NCHUNK = 80
EPT = NCHUNK * K
GRPS = (48, 32)
GMAX = max(GRPS)
ROWS_PT = NP // TILES


@functools.cache
def _make_sc_agg():
    mesh = plsc.VectorSubcoreMesh(core_axis_name="c", subcore_axis_name="s")
    return pl.kernel(
        _sc_agg_body,
        out_type=(
            jax.ShapeDtypeStruct((NP, DH), jnp.float32),
            jax.ShapeDtypeStruct((NP, DH), jnp.float32),
        ),
        mesh=mesh,
        scratch_types=[
            pltpu.VMEM((EPT,), jnp.int32),
            pltpu.VMEM((GMAX, K), jnp.int32),
            pltpu.VMEM((2 * K, DH), jnp.float32),
            pltpu.VMEM_SHARED((NP, DH), jnp.float32),
            pltpu.SemaphoreType.DMA,
            pltpu.SemaphoreType.DMA,
        ],
    )


def _sc_agg_body(h_lo, h_hi, srcp, dstp, m_lo, m_hi,
                 src_v, dst_v, ring, acc, gs0, gs1):
    gsem = (gs0, gs1)
    c = lax.axis_index("c")
    s = lax.axis_index("s")
    base = s * ROWS_PT

    @pl.when(c == 0)
    def _():
        pltpu.sync_copy(h_lo.at[pl.ds(base, ROWS_PT)], acc.at[pl.ds(base, ROWS_PT)])

    @pl.when(c == 1)
    def _():
        pltpu.sync_copy(h_hi.at[pl.ds(base, ROWS_PT)], acc.at[pl.ds(base, ROWS_PT)])

    plsc.subcore_barrier()

    def edges(h_ref):
        pltpu.sync_copy(srcp.at[s], src_v)

        def slot_buf(slot):
            return ring.at[pl.ds(slot * K, K)]

        def gather(chunk, slot):
            pltpu.async_copy(
                h_ref.at[src_v.at[pl.ds(chunk * K, K)]], slot_buf(slot), gsem[slot])

        def wait_gather(chunk, slot):
            pltpu.make_async_copy(
                h_ref.at[src_v.at[pl.ds(chunk * K, K)]], slot_buf(slot),
                gsem[slot]).wait()

        gather(0, 0)
        goff = 0
        for gsize in GRPS:
            pairs = gsize // 2
            pltpu.sync_copy(dstp.at[s, pl.ds(goff, gsize)], dst_v.at[pl.ds(0, gsize)])

            def body(j, carry, goff=goff, pairs=pairs):
                a = goff + 2 * j
                b = a + 1
                gather(b, 1)
                wait_gather(a, 0)
                pltpu.sync_copy(slot_buf(0), acc.at[dst_v.at[2 * j]], add=True)

                @pl.when(a + 2 < NCHUNK)
                def _():
                    gather(a + 2, 0)

                wait_gather(b, 1)
                pltpu.sync_copy(slot_buf(1), acc.at[dst_v.at[2 * j + 1]], add=True)
                return carry

            lax.fori_loop(0, pairs, body, 0)
            goff += gsize

    @pl.when(c == 0)
    def _():
        edges(h_lo)

    @pl.when(c == 1)
    def _():
        edges(h_hi)

    plsc.subcore_barrier()

    @pl.when(c == 0)
    def _():
        pltpu.sync_copy(acc.at[pl.ds(base, ROWS_PT)], m_lo.at[pl.ds(base, ROWS_PT)])

    @pl.when(c == 1)
    def _():
        pltpu.sync_copy(acc.at[pl.ds(base, ROWS_PT)], m_hi.at[pl.ds(base, ROWS_PT)])


def _mlp_body_split(mlo, mhi, w1, b1, w2, b2, olo, ohi):
    m = jnp.concatenate([mlo[...], mhi[...]], axis=1)
    t = jnp.maximum(jnp.dot(m, w1[...], preferred_element_type=jnp.float32) + b1[...], 0.0)
    h = jnp.dot(t, w2[...], preferred_element_type=jnp.float32) + b2[...]
    olo[...] = h[:, :DH]
    ohi[...] = h[:, DH:]


def _mlp_body_final(mlo, mhi, w1, b1, w2, b2, out):
    m = jnp.concatenate([mlo[...], mhi[...]], axis=1)
    t = jnp.maximum(jnp.dot(m, w1[...], preferred_element_type=jnp.float32) + b1[...], 0.0)
    out[...] = jnp.dot(t, w2[...], preferred_element_type=jnp.float32) + b2[...]


_BM = 1264
_W_SPECS = [
    pl.BlockSpec((2 * DH, 2 * DH), lambda i: (0, 0)),
    pl.BlockSpec((1, 2 * DH), lambda i: (0, 0)),
    pl.BlockSpec((2 * DH, 2 * DH), lambda i: (0, 0)),
    pl.BlockSpec((1, 2 * DH), lambda i: (0, 0)),
]

_mlp_split = pl.pallas_call(
    _mlp_body_split,
    grid=(NP // _BM,),
    in_specs=[
        pl.BlockSpec((_BM, DH), lambda i: (i, 0)),
        pl.BlockSpec((_BM, DH), lambda i: (i, 0)),
        *_W_SPECS,
    ],
    out_specs=[
        pl.BlockSpec((_BM, DH), lambda i: (i, 0)),
        pl.BlockSpec((_BM, DH), lambda i: (i, 0)),
    ],
    out_shape=[
        jax.ShapeDtypeStruct((NP, DH), jnp.float32),
        jax.ShapeDtypeStruct((NP, DH), jnp.float32),
    ],
)

_BMF = 1000
_mlp_final = pl.pallas_call(
    _mlp_body_final,
    grid=(N // _BMF,),
    in_specs=[
        pl.BlockSpec((_BMF, DH), lambda i: (i, 0)),
        pl.BlockSpec((_BMF, DH), lambda i: (i, 0)),
        *_W_SPECS,
    ],
    out_specs=pl.BlockSpec((_BMF, 2 * DH), lambda i: (i, 0)),
    out_shape=jax.ShapeDtypeStruct((N, 2 * DH), jnp.float32),
)


def kernel(x, edge_index, W1_0, b1_0, W2_0, b2_0, W1_1, b1_1, W2_1, b2_1,
           W1_2, b1_2, W2_2, b2_2):
    pad = TILES * EPT - E
    srcp = jnp.concatenate(
        [edge_index[0], jnp.zeros((pad,), jnp.int32)]).reshape(TILES, EPT)
    dstp = jnp.concatenate(
        [edge_index[1], jnp.full((pad,), N, jnp.int32)]).reshape(TILES, NCHUNK, K)

    h_lo = jnp.zeros((NP, DH), jnp.float32).at[:N].set(x[:, :DH])
    h_hi = jnp.zeros((NP, DH), jnp.float32).at[:N].set(x[:, DH:])

    params = [
        (W1_0, b1_0.reshape(1, -1), W2_0, b2_0.reshape(1, -1)),
        (W1_1, b1_1.reshape(1, -1), W2_1, b2_1.reshape(1, -1)),
        (W1_2, b1_2.reshape(1, -1), W2_2, b2_2.reshape(1, -1)),
    ]
    for i, (W1, b1, W2, b2) in enumerate(params):
        m_lo, m_hi = _make_sc_agg()(h_lo, h_hi, srcp, dstp)
        if i < 2:
            h_lo, h_hi = _mlp_split(m_lo, m_hi, W1, b1, W2, b2)
        else:
            return _mlp_final(m_lo, m_hi, W1, b1, W2, b2)
